# baseline (speedup 1.0000x reference)
import jax
import jax.numpy as jnp
from jax import lax
from jax.experimental import pallas as pl
from jax.experimental.pallas import tpu as pltpu
from jax.experimental.pallas import tpu_sc as plsc

B, T, H = 16, 4096, 128
G = 2048
NC, NS = 2, 16
RPB = 2
GR = G // (RPB * 2)
CHUNK = 128
NCHUNKS = T // CHUNK
HV = H // 16


def _sc_body(x_hbm, gb_hbm, out_hbm, gb_v, stage_v, acc_v,
             sem_gb, sem_x0, sem_x1, sem_o):
    cid = lax.axis_index("c")
    sid = lax.axis_index("s")
    wid = sid * NC + cid
    b = wid // 2
    r_base = (wid % 2) * RPB

    cp_gb = pltpu.make_async_copy(gb_hbm.at[b], gb_v, sem_gb)
    cp_gb.start()
    cp_gb.wait()

    zero16 = jnp.zeros((16,), jnp.float32)
    one16 = jnp.full((16,), 1.0, jnp.float32)

    bounds = [(r_base + i) * GR for i in range(RPB + 1)]

    def bc(i, carry):
        gv = gb_v[pl.ds(i * 16, 16)]
        return tuple(carry[i2] + jnp.where(gv < bounds[i2], 1.0, 0.0)
                     for i2 in range(RPB + 1))

    cvs = lax.fori_loop(0, T // 16, bc, (zero16,) * (RPB + 1))
    ts = []
    for cv in cvs:
        s = cv[0]
        for u in range(1, 16):
            s = s + cv[u]
        ts.append(s.astype(jnp.int32))

    for ri in range(RPB):
        g0 = bounds[ri]
        g1 = bounds[ri + 1]
        t0 = ts[ri]
        t1 = ts[ri + 1]
        c_lo = t0 // CHUNK
        c_hi = (t1 + (CHUNK - 1)) // CHUNK
        odd = (c_hi - c_lo) % 2 == 1
        c_hi = jnp.where(jnp.logical_and(odd, c_hi < NCHUNKS), c_hi + 1, c_hi)
        c_lo = jnp.where((c_hi - c_lo) % 2 == 1, c_lo - 1, c_lo)
        npairs = (c_hi - c_lo) // 2

        def start_dma(c, sbase, sem):
            pltpu.make_async_copy(
                x_hbm.at[b, pl.ds(c * CHUNK, CHUNK), :],
                stage_v.at[pl.ds(sbase, CHUNK), :], sem).start()

        def wait_dma(c, sbase, sem):
            pltpu.make_async_copy(
                x_hbm.at[b, pl.ds(c * CHUNK, CHUNK), :],
                stage_v.at[pl.ds(sbase, CHUNK), :], sem).wait()

        @pl.when(npairs > 0)
        def _(c_lo=c_lo):
            start_dma(c_lo, 0, sem_x0)
            start_dma(c_lo + 1, CHUNK, sem_x1)

        def zero_rows(a, bnd):
            def zb(g, _):
                row = acc_v.at[g - g0]
                for k in range(HV):
                    row[pl.ds(k * 16, 16)] = zero16
                return 0
            lax.fori_loop(a, bnd, zb, 0)

        def flush(g_cur, cnt, acc, lastg):
            zero_rows(jnp.maximum(lastg + 1, g0), jnp.minimum(g_cur, g1))
            in_r = jnp.logical_and(g_cur >= g0, g_cur < g1)
            idx = jnp.where(in_r, g_cur - g0, GR)
            inv = one16 / jnp.broadcast_to(cnt, (16,))
            row = acc_v.at[idx]
            for k in range(HV):
                row[pl.ds(k * 16, 16)] = acc[k] * inv

        def process(c, sbase0, carry):
            def sub_body(jj, carry):
                g_cur, cnt, acc, lastg = carry
                gvec = gb_v[pl.ds(c * CHUNK + jj * 16, 16)]
                for u in range(16):
                    g = gvec[u]
                    same = g == g_cur

                    @pl.when(jnp.logical_not(same))
                    def _(g_cur=g_cur, cnt=cnt, acc=acc, lastg=lastg):
                        flush(g_cur, cnt, acc, lastg)

                    lastg = jnp.where(
                        same, lastg,
                        jnp.minimum(jnp.maximum(g_cur, lastg), g1))
                    keep = jnp.where(same, 1.0, 0.0)
                    keepv = jnp.broadcast_to(keep, (16,))
                    srow = stage_v.at[sbase0 + jj * 16 + u]
                    xrow = [srow[pl.ds(k * 16, 16)] for k in range(HV)]
                    acc = tuple(
                        acc[k] * keepv + xrow[k] for k in range(HV))
                    cnt = cnt * keep + 1.0
                    g_cur = g
                return g_cur, cnt, acc, lastg

            return lax.fori_loop(0, CHUNK // 16, sub_body, carry)

        def pair_body(p, carry):
            c = c_lo + 2 * p
            wait_dma(c, 0, sem_x0)
            carry = process(c, 0, carry)

            @pl.when(c + 2 < c_hi)
            def _(c=c):
                start_dma(c + 2, 0, sem_x0)

            wait_dma(c + 1, CHUNK, sem_x1)
            carry = process(c + 1, CHUNK, carry)

            @pl.when(c + 3 < c_hi)
            def _(c=c):
                start_dma(c + 3, CHUNK, sem_x1)

            return carry

        init = (jnp.int32(-1), jnp.float32(0.0),
                tuple(jnp.zeros((16,), jnp.float32) for _ in range(HV)),
                jnp.int32(g0 - 1))
        g_cur, cnt, acc, lastg = lax.fori_loop(0, npairs, pair_body, init)
        flush(g_cur, cnt, acc, lastg)
        lastg = jnp.minimum(jnp.maximum(g_cur, lastg), g1)
        zero_rows(jnp.maximum(lastg + 1, g0), g1)

        cp_o = pltpu.make_async_copy(
            acc_v.at[pl.ds(0, GR), :], out_hbm.at[b, pl.ds(g0, GR), :], sem_o)
        cp_o.start()
        cp_o.wait()


@jax.jit
def _run(x, gb):
    mesh = plsc.VectorSubcoreMesh(core_axis_name="c", subcore_axis_name="s")
    f = pl.kernel(
        _sc_body,
        out_type=jax.ShapeDtypeStruct((B, G, H), jnp.float32),
        mesh=mesh,
        scratch_types=[
            pltpu.VMEM((T,), jnp.int32),
            pltpu.VMEM((2 * CHUNK, H), jnp.float32),
            pltpu.VMEM((GR + 1, H), jnp.float32),
            pltpu.SemaphoreType.DMA,
            pltpu.SemaphoreType.DMA,
            pltpu.SemaphoreType.DMA,
            pltpu.SemaphoreType.DMA,
        ],
    )
    return f(x, gb)


def kernel(x, group_by, agg_step):
    return _run(x, group_by.astype(jnp.int32))


# speedup vs baseline: 1.3940x; 1.3940x over previous
import jax
import jax.numpy as jnp
from jax import lax
from jax.experimental import pallas as pl
from jax.experimental.pallas import tpu as pltpu
from jax.experimental.pallas import tpu_sc as plsc

B, T, H = 16, 4096, 128
G = 2048
NC, NS = 2, 16
RPB = 2
GR = G // (RPB * 2)
CHUNK = 128
NCHUNKS = T // CHUNK
HV = H // 16


def _sc_body(x_hbm, gb_hbm, z_hbm, out_hbm, gb_v, stage_v, acc_v,
             sem_gb, sem_x0, sem_x1, sem_o, sem_z):
    cid = lax.axis_index("c")
    sid = lax.axis_index("s")
    wid = sid * NC + cid
    b = wid // 2
    r_base = (wid % 2) * RPB

    cp_gb = pltpu.make_async_copy(gb_hbm.at[b], gb_v, sem_gb)
    cp_gb.start()
    cp_gb.wait()

    zero16 = jnp.zeros((16,), jnp.float32)
    one16 = jnp.full((16,), 1.0, jnp.float32)

    bounds = [(r_base + i) * GR for i in range(RPB + 1)]

    def bc(i, carry):
        gv = gb_v[pl.ds(i * 16, 16)]
        return tuple(carry[i2] + jnp.where(gv < bounds[i2], 1.0, 0.0)
                     for i2 in range(RPB + 1))

    cvs = lax.fori_loop(0, T // 16, bc, (zero16,) * (RPB + 1))
    ts = []
    for cv in cvs:
        s = cv[0]
        for u in range(1, 16):
            s = s + cv[u]
        ts.append(s.astype(jnp.int32))

    for ri in range(RPB):
        g0 = bounds[ri]
        g1 = bounds[ri + 1]
        t0 = ts[ri]
        t1 = ts[ri + 1]
        c_lo = t0 // CHUNK
        c_hi = (t1 + (CHUNK - 1)) // CHUNK
        odd = (c_hi - c_lo) % 2 == 1
        c_hi = jnp.where(jnp.logical_and(odd, c_hi < NCHUNKS), c_hi + 1, c_hi)
        c_lo = jnp.where((c_hi - c_lo) % 2 == 1, c_lo - 1, c_lo)
        npairs = (c_hi - c_lo) // 2

        def start_dma(c, sbase, sem):
            pltpu.make_async_copy(
                x_hbm.at[b, pl.ds(c * CHUNK, CHUNK), :],
                stage_v.at[pl.ds(sbase, CHUNK), :], sem).start()

        def wait_dma(c, sbase, sem):
            pltpu.make_async_copy(
                x_hbm.at[b, pl.ds(c * CHUNK, CHUNK), :],
                stage_v.at[pl.ds(sbase, CHUNK), :], sem).wait()

        cp_z = pltpu.make_async_copy(z_hbm, acc_v, sem_z)
        cp_z.start()

        @pl.when(npairs > 0)
        def _(c_lo=c_lo):
            start_dma(c_lo, 0, sem_x0)
            start_dma(c_lo + 1, CHUNK, sem_x1)

        cp_z.wait()

        def flush(g_cur, cnt, acc):
            in_r = jnp.logical_and(g_cur >= g0, g_cur < g1)
            idx = jnp.where(in_r, g_cur - g0, GR)
            inv = one16 / jnp.broadcast_to(cnt, (16,))
            row = acc_v.at[idx]
            for k in range(HV):
                row[pl.ds(k * 16, 16)] = acc[k] * inv

        def process(c, sbase0, carry):
            def sub_body(jj, carry):
                g_cur, cnt, acc = carry
                gvec = gb_v[pl.ds(c * CHUNK + jj * 16, 16)]
                for u in range(16):
                    g = gvec[u]
                    same = g == g_cur

                    @pl.when(jnp.logical_not(same))
                    def _(g_cur=g_cur, cnt=cnt, acc=acc):
                        flush(g_cur, cnt, acc)

                    keep = jnp.where(same, 1.0, 0.0)
                    keepv = jnp.broadcast_to(keep, (16,))
                    srow = stage_v.at[sbase0 + jj * 16 + u]
                    xrow = [srow[pl.ds(k * 16, 16)] for k in range(HV)]
                    acc = tuple(
                        acc[k] * keepv + xrow[k] for k in range(HV))
                    cnt = cnt * keep + 1.0
                    g_cur = g
                return g_cur, cnt, acc

            return lax.fori_loop(0, CHUNK // 16, sub_body, carry)

        def pair_body(p, carry):
            c = c_lo + 2 * p
            wait_dma(c, 0, sem_x0)
            carry = process(c, 0, carry)

            @pl.when(c + 2 < c_hi)
            def _(c=c):
                start_dma(c + 2, 0, sem_x0)

            wait_dma(c + 1, CHUNK, sem_x1)
            carry = process(c + 1, CHUNK, carry)

            @pl.when(c + 3 < c_hi)
            def _(c=c):
                start_dma(c + 3, CHUNK, sem_x1)

            return carry

        init = (jnp.int32(-1), jnp.float32(0.0),
                tuple(jnp.zeros((16,), jnp.float32) for _ in range(HV)))
        g_cur, cnt, acc = lax.fori_loop(0, npairs, pair_body, init)
        flush(g_cur, cnt, acc)

        cp_o = pltpu.make_async_copy(
            acc_v.at[pl.ds(0, GR), :], out_hbm.at[b, pl.ds(g0, GR), :], sem_o)
        cp_o.start()
        cp_o.wait()


@jax.jit
def _run(x, gb):
    mesh = plsc.VectorSubcoreMesh(core_axis_name="c", subcore_axis_name="s")
    f = pl.kernel(
        _sc_body,
        out_type=jax.ShapeDtypeStruct((B, G, H), jnp.float32),
        mesh=mesh,
        scratch_types=[
            pltpu.VMEM((T,), jnp.int32),
            pltpu.VMEM((2 * CHUNK, H), jnp.float32),
            pltpu.VMEM((GR + 1, H), jnp.float32),
            pltpu.SemaphoreType.DMA,
            pltpu.SemaphoreType.DMA,
            pltpu.SemaphoreType.DMA,
            pltpu.SemaphoreType.DMA,
            pltpu.SemaphoreType.DMA,
        ],
    )
    return f(x, gb, jnp.zeros((GR + 1, H), jnp.float32))


def kernel(x, group_by, agg_step):
    return _run(x, group_by.astype(jnp.int32))


# speedup vs baseline: 1.6242x; 1.1652x over previous
import jax
import jax.numpy as jnp
from jax import lax
from jax.experimental import pallas as pl
from jax.experimental.pallas import tpu as pltpu
from jax.experimental.pallas import tpu_sc as plsc

B, T, H = 16, 4096, 128
G = 2048
NC, NS = 2, 16
RPB = 2
GR = G // (RPB * 2)
CHUNK = 128
NCHUNKS = T // CHUNK
HV = H // 16


def _sc_body(x_hbm, gb_hbm, out_hbm, gb_v, stage_v, acc_v,
             sem_gb, sem_x0, sem_x1, sem_o):
    cid = lax.axis_index("c")
    sid = lax.axis_index("s")
    wid = sid * NC + cid
    b = wid // 2
    r_base = (wid % 2) * RPB

    cp_gb = pltpu.make_async_copy(gb_hbm.at[b], gb_v, sem_gb)
    cp_gb.start()
    cp_gb.wait()

    zero16 = jnp.zeros((16,), jnp.float32)
    one16 = jnp.full((16,), 1.0, jnp.float32)

    bounds = [(r_base + i) * GR for i in range(RPB + 1)]

    def bc(i, carry):
        gv = gb_v[pl.ds(i * 16, 16)]
        return tuple(carry[i2] + jnp.where(gv < bounds[i2], 1.0, 0.0)
                     for i2 in range(RPB + 1))

    cvs = lax.fori_loop(0, T // 16, bc, (zero16,) * (RPB + 1))
    ts = []
    for cv in cvs:
        s = cv[0]
        for u in range(1, 16):
            s = s + cv[u]
        ts.append(s.astype(jnp.int32))

    for ri in range(RPB):
        g0 = bounds[ri]
        g1 = bounds[ri + 1]
        t0 = ts[ri]
        t1 = ts[ri + 1]
        c_lo = t0 // CHUNK
        c_hi = (t1 + (CHUNK - 1)) // CHUNK
        odd = (c_hi - c_lo) % 2 == 1
        c_hi = jnp.where(jnp.logical_and(odd, c_hi < NCHUNKS), c_hi + 1, c_hi)
        c_lo = jnp.where((c_hi - c_lo) % 2 == 1, c_lo - 1, c_lo)
        npairs = (c_hi - c_lo) // 2

        def start_dma(c, sbase, sem):
            pltpu.make_async_copy(
                x_hbm.at[b, pl.ds(c * CHUNK, CHUNK), :],
                stage_v.at[pl.ds(sbase, CHUNK), :], sem).start()

        def wait_dma(c, sbase, sem):
            pltpu.make_async_copy(
                x_hbm.at[b, pl.ds(c * CHUNK, CHUNK), :],
                stage_v.at[pl.ds(sbase, CHUNK), :], sem).wait()

        @pl.when(npairs > 0)
        def _(c_lo=c_lo):
            start_dma(c_lo, 0, sem_x0)
            start_dma(c_lo + 1, CHUNK, sem_x1)

        def za(r, _):
            row = acc_v.at[r]
            for k in range(HV):
                row[pl.ds(k * 16, 16)] = zero16
            return 0
        lax.fori_loop(0, GR + 1, za, 0)

        def flush(g_cur, cnt, acc):
            in_r = jnp.logical_and(g_cur >= g0, g_cur < g1)
            idx = jnp.where(in_r, g_cur - g0, GR)
            inv = one16 / jnp.broadcast_to(cnt, (16,))
            row = acc_v.at[idx]
            for k in range(HV):
                row[pl.ds(k * 16, 16)] = acc[k] * inv

        def process(c, sbase0, carry):
            def sub_body(jj, carry):
                g_cur, cnt, acc = carry
                gvec = gb_v[pl.ds(c * CHUNK + jj * 16, 16)]
                for u in range(16):
                    g = gvec[u]
                    same = g == g_cur

                    @pl.when(jnp.logical_not(same))
                    def _(g_cur=g_cur, cnt=cnt, acc=acc):
                        flush(g_cur, cnt, acc)

                    keep = jnp.where(same, 1.0, 0.0)
                    keepv = jnp.broadcast_to(keep, (16,))
                    srow = stage_v.at[sbase0 + jj * 16 + u]
                    xrow = [srow[pl.ds(k * 16, 16)] for k in range(HV)]
                    acc = tuple(
                        acc[k] * keepv + xrow[k] for k in range(HV))
                    cnt = cnt * keep + 1.0
                    g_cur = g
                return g_cur, cnt, acc

            return lax.fori_loop(0, CHUNK // 16, sub_body, carry)

        def pair_body(p, carry):
            c = c_lo + 2 * p
            wait_dma(c, 0, sem_x0)
            carry = process(c, 0, carry)

            @pl.when(c + 2 < c_hi)
            def _(c=c):
                start_dma(c + 2, 0, sem_x0)

            wait_dma(c + 1, CHUNK, sem_x1)
            carry = process(c + 1, CHUNK, carry)

            @pl.when(c + 3 < c_hi)
            def _(c=c):
                start_dma(c + 3, CHUNK, sem_x1)

            return carry

        init = (jnp.int32(-1), jnp.float32(0.0),
                tuple(jnp.zeros((16,), jnp.float32) for _ in range(HV)))
        g_cur, cnt, acc = lax.fori_loop(0, npairs, pair_body, init)
        flush(g_cur, cnt, acc)

        cp_o = pltpu.make_async_copy(
            acc_v.at[pl.ds(0, GR), :], out_hbm.at[b, pl.ds(g0, GR), :], sem_o)
        cp_o.start()
        cp_o.wait()


@jax.jit
def _run(x, gb):
    mesh = plsc.VectorSubcoreMesh(core_axis_name="c", subcore_axis_name="s")
    f = pl.kernel(
        _sc_body,
        out_type=jax.ShapeDtypeStruct((B, G, H), jnp.float32),
        mesh=mesh,
        scratch_types=[
            pltpu.VMEM((T,), jnp.int32),
            pltpu.VMEM((2 * CHUNK, H), jnp.float32),
            pltpu.VMEM((GR + 1, H), jnp.float32),
            pltpu.SemaphoreType.DMA,
            pltpu.SemaphoreType.DMA,
            pltpu.SemaphoreType.DMA,
            pltpu.SemaphoreType.DMA,
        ],
    )
    return f(x, gb)


def kernel(x, group_by, agg_step):
    return _run(x, group_by.astype(jnp.int32))
